# Initial kernel scaffold; baseline (speedup 1.0000x reference)
import jax
import jax.numpy as jnp
from jax.experimental import pallas as pl


def kernel(node_repr, edge_repr, nb_node_d1, nb_edge_d1, nb_node_d2, nb_edge_d2, nb_node_d4, nb_edge_d4, nb_node_d8, nb_edge_d8, W_self, W_deg1, W_deg2, W_deg4, W_deg8, W_deg16, bias):
    raise NotImplementedError("write your pallas kernel here")


# speedup vs baseline: 2.3227x; 2.3227x over previous
import functools

import jax
import jax.numpy as jnp
from jax import lax
from jax.experimental import pallas as pl
from jax.experimental.pallas import tpu as pltpu
from jax.experimental.pallas import tpu_sc as plsc

N_NODES = 100000
NODE_SIZE = 128
EDGE_SIZE = 16
OUT_SIZE = 128
GROUP = 25000
DEGS = (1, 2, 4, 8)
N_EDGES = 375000
EPS = 1e-5

NC = 2
NS = 16
NW = NC * NS
LANES = 16
BLK = 128


def _ceil_div(a, b):
    return (a + b - 1) // b


def _sc_body(node_hbm, edge_hbm,
             ni1, ei1, ni2, ei2, ni4, ei4, ni8, ei8,
             nsum_hbm, esum_hbm,
             nidx_v, eidx_v, nrows_v, erows_v, nout_v, eout_v,
             sem_n, sem_e):
    wid = lax.axis_index("s") * NC + lax.axis_index("c")
    nidx_all = (ni1, ni2, ni4, ni8)
    eidx_all = (ei1, ei2, ei4, ei8)
    for b, d in enumerate(DEGS):
        nidx_hbm = nidx_all[b]
        eidx_hbm = eidx_all[b]
        nblk = _ceil_div(GROUP * d, BLK)
        last_gbase = GROUP * d - BLK
        bd = BLK // d
        obase0 = b * GROUP

        def body(i, _, d=d, bd=bd, nidx_hbm=nidx_hbm, eidx_hbm=eidx_hbm,
                 last_gbase=last_gbase, obase0=obase0):
            blk_id = wid + i * NW
            gbase = pl.multiple_of(jnp.minimum(blk_id * BLK, last_gbase), 8)
            obase = pl.multiple_of(obase0 + gbase // d, 8)
            pltpu.sync_copy(nidx_hbm.at[pl.ds(gbase, BLK)], nidx_v)
            pltpu.sync_copy(eidx_hbm.at[pl.ds(gbase, BLK)], eidx_v)
            if d == 1:
                cp_n = pltpu.async_copy(node_hbm.at[nidx_v], nout_v, sem_n)
                cp_e = pltpu.async_copy(edge_hbm.at[eidx_v], eout_v, sem_e)
                cp_n.wait()
                cp_e.wait()
            else:
                cp_n = pltpu.async_copy(node_hbm.at[nidx_v], nrows_v, sem_n)
                cp_e = pltpu.async_copy(edge_hbm.at[eidx_v], erows_v, sem_e)
                cp_n.wait()
                cp_e.wait()

                def acc_row(r, _):
                    base = r * d
                    for c in range(NODE_SIZE // LANES):
                        s = nrows_v[base, pl.ds(c * LANES, LANES)]
                        for j in range(1, d):
                            s = s + nrows_v[base + j, pl.ds(c * LANES, LANES)]
                        nout_v[r, pl.ds(c * LANES, LANES)] = s
                    se = erows_v[base, :]
                    for j in range(1, d):
                        se = se + erows_v[base + j, :]
                    eout_v[r, :] = se
                    return 0

                lax.fori_loop(0, bd, acc_row, 0, unroll=False)
            pltpu.sync_copy(nout_v.at[pl.ds(0, bd)],
                            nsum_hbm.at[pl.ds(obase, bd)])
            pltpu.sync_copy(eout_v.at[pl.ds(0, bd)],
                            esum_hbm.at[pl.ds(obase, bd)])
            return 0

        cnt = (nblk - wid + NW - 1) // NW
        lax.fori_loop(0, cnt, body, 0, unroll=False)


def _sc_gather_sums(node_repr, edge_repr, nidx, eidx):
    mesh = plsc.VectorSubcoreMesh(core_axis_name="c", subcore_axis_name="s")
    out_type = (
        jax.ShapeDtypeStruct((4 * GROUP, NODE_SIZE), jnp.float32),
        jax.ShapeDtypeStruct((4 * GROUP, EDGE_SIZE), jnp.float32),
    )
    scratch = [
        pltpu.VMEM((BLK,), jnp.int32),
        pltpu.VMEM((BLK,), jnp.int32),
        pltpu.VMEM((BLK, NODE_SIZE), jnp.float32),
        pltpu.VMEM((BLK, EDGE_SIZE), jnp.float32),
        pltpu.VMEM((BLK, NODE_SIZE), jnp.float32),
        pltpu.VMEM((BLK, EDGE_SIZE), jnp.float32),
        pltpu.SemaphoreType.DMA,
        pltpu.SemaphoreType.DMA,
    ]
    fn = pl.kernel(_sc_body, out_type=out_type, mesh=mesh,
                   scratch_types=scratch,
                   compiler_params=pltpu.CompilerParams(
                       use_tc_tiling_on_sc=False))
    return fn(node_repr, edge_repr,
              nidx[0], eidx[0], nidx[1], eidx[1],
              nidx[2], eidx[2], nidx[3], eidx[3])


R1 = 1000
NB1 = 4 * GROUP // R1
BPB = GROUP // R1

R2 = 2000
NB2 = 4 * GROUP // R2


def _tc_act_body(node_ref, nsum_ref, esum_ref, wself_ref, wn_ref, we_ref,
                 bias_ref, act_ref, s1_ref, s2_ref):
    g = pl.program_id(0)
    act = jnp.dot(node_ref[...], wself_ref[...],
                  preferred_element_type=jnp.float32)
    act = act + jnp.dot(nsum_ref[...], wn_ref[0],
                        preferred_element_type=jnp.float32)
    act = act + jnp.dot(esum_ref[...], we_ref[0],
                        preferred_element_type=jnp.float32)
    act = act + bias_ref[...]
    act_ref[...] = act
    p = jnp.sum(act.reshape(R1 // 8, 8, OUT_SIZE), axis=0)
    q = jnp.sum((act * act).reshape(R1 // 8, 8, OUT_SIZE), axis=0)

    @pl.when(g == 0)
    def _():
        s1_ref[...] = p
        s2_ref[...] = q

    @pl.when(g > 0)
    def _():
        s1_ref[...] += p
        s2_ref[...] += q


def _tc_act(node_repr, nsum, esum, W_self, wn_stack, we_stack, bias):
    return pl.pallas_call(
        _tc_act_body,
        grid=(NB1,),
        in_specs=[
            pl.BlockSpec((R1, NODE_SIZE), lambda g: (g, 0)),
            pl.BlockSpec((R1, NODE_SIZE), lambda g: (g, 0)),
            pl.BlockSpec((R1, EDGE_SIZE), lambda g: (g, 0)),
            pl.BlockSpec((NODE_SIZE, OUT_SIZE), lambda g: (0, 0)),
            pl.BlockSpec((1, NODE_SIZE, OUT_SIZE), lambda g: (g // BPB, 0, 0)),
            pl.BlockSpec((1, EDGE_SIZE, OUT_SIZE), lambda g: (g // BPB, 0, 0)),
            pl.BlockSpec((1, OUT_SIZE), lambda g: (0, 0)),
        ],
        out_specs=[
            pl.BlockSpec((R1, OUT_SIZE), lambda g: (g, 0)),
            pl.BlockSpec((8, OUT_SIZE), lambda g: (0, 0)),
            pl.BlockSpec((8, OUT_SIZE), lambda g: (0, 0)),
        ],
        out_shape=[
            jax.ShapeDtypeStruct((4 * GROUP, OUT_SIZE), jnp.float32),
            jax.ShapeDtypeStruct((8, OUT_SIZE), jnp.float32),
            jax.ShapeDtypeStruct((8, OUT_SIZE), jnp.float32),
        ],
    )(node_repr, nsum, esum, W_self, wn_stack, we_stack, bias)


def _tc_norm_body(act_ref, mu_ref, inv_ref, out_ref):
    out_ref[...] = jnp.maximum(
        (act_ref[...] - mu_ref[...]) * inv_ref[...], 0.0)


def _tc_norm(act, mu, inv):
    return pl.pallas_call(
        _tc_norm_body,
        grid=(NB2,),
        in_specs=[
            pl.BlockSpec((R2, OUT_SIZE), lambda g: (g, 0)),
            pl.BlockSpec((1, OUT_SIZE), lambda g: (0, 0)),
            pl.BlockSpec((1, OUT_SIZE), lambda g: (0, 0)),
        ],
        out_specs=pl.BlockSpec((R2, OUT_SIZE), lambda g: (g, 0)),
        out_shape=jax.ShapeDtypeStruct((4 * GROUP, OUT_SIZE), jnp.float32),
    )(act, mu, inv)


def kernel(node_repr, edge_repr, nb_node_d1, nb_edge_d1, nb_node_d2,
           nb_edge_d2, nb_node_d4, nb_edge_d4, nb_node_d8, nb_edge_d8,
           W_self, W_deg1, W_deg2, W_deg4, W_deg8, W_deg16, bias):
    nidx = tuple(a.reshape(-1).astype(jnp.int32)
                 for a in (nb_node_d1, nb_node_d2, nb_node_d4, nb_node_d8))
    eidx = tuple(a.reshape(-1).astype(jnp.int32)
                 for a in (nb_edge_d1, nb_edge_d2, nb_edge_d4, nb_edge_d8))
    nsum, esum = _sc_gather_sums(node_repr, edge_repr, nidx, eidx)

    wn_stack = jnp.stack([W[:NODE_SIZE] for W in
                          (W_deg1, W_deg2, W_deg4, W_deg8)])
    we_stack = jnp.stack([W[NODE_SIZE:] for W in
                          (W_deg1, W_deg2, W_deg4, W_deg8)])
    act, s1, s2 = _tc_act(node_repr, nsum, esum, W_self, wn_stack, we_stack,
                          bias)
    n = jnp.float32(4 * GROUP)
    mu = jnp.sum(s1, axis=0, keepdims=True) / n
    var = jnp.sum(s2, axis=0, keepdims=True) / n - mu * mu
    inv = lax.rsqrt(var + EPS)
    return _tc_norm(act, mu, inv)


# speedup vs baseline: 2.8531x; 1.2283x over previous
import functools

import jax
import jax.numpy as jnp
from jax import lax
from jax.experimental import pallas as pl
from jax.experimental.pallas import tpu as pltpu
from jax.experimental.pallas import tpu_sc as plsc

N_NODES = 100000
NODE_SIZE = 128
EDGE_SIZE = 16
OUT_SIZE = 128
GROUP = 25000
DEGS = (1, 2, 4, 8)
N_EDGES = 375000
EPS = 1e-5

NC = 2
NS = 16
NW = NC * NS
LANES = 16
BLK = 128


def _ceil_div(a, b):
    return (a + b - 1) // b


NBLKS = tuple(_ceil_div(GROUP * d, BLK) for d in DEGS)
PERS = tuple(_ceil_div(nb, NW) for nb in NBLKS)
GP = max(nb * BLK // d for nb, d in zip(NBLKS, DEGS))


def _sc_body(node_hbm, edge_hbm,
             ni1, ei1, ni2, ei2, ni4, ei4, ni8, ei8,
             nsum_hbm, esum_hbm,
             nidx_v, eidx_v,
             nrows, erows, nout, eout,
             gsem_n, gsem_e, osem_n, osem_e):
    wid = lax.axis_index("s") * NC + lax.axis_index("c")
    nidx_all = (ni1, ni2, ni4, ni8)
    eidx_all = (ei1, ei2, ei4, ei8)

    for b, d in enumerate(DEGS):
        nidx_hbm = nidx_all[b]
        eidx_hbm = eidx_all[b]
        nblk = NBLKS[b]
        per = PERS[b]
        bd = BLK // d

        w0 = jnp.minimum(wid * per, nblk - per)
        pltpu.sync_copy(nidx_hbm.at[pl.ds(w0, per)], nidx_v.at[pl.ds(0, per)])
        pltpu.sync_copy(eidx_hbm.at[pl.ds(w0, per)], eidx_v.at[pl.ds(0, per)])

        def issue_g(j, p):
            cn = pltpu.async_copy(node_hbm.at[nidx_v.at[j]], nrows.at[p],
                                  gsem_n.at[p])
            ce = pltpu.async_copy(edge_hbm.at[eidx_v.at[j]], erows.at[p],
                                  gsem_e.at[p])
            return cn, ce

        def wait_g(cps):
            cps[0].wait()
            cps[1].wait()

        def acc(j, p, d=d, bd=bd):
            def acc_row(r, _):
                base = r * d
                for c in range(NODE_SIZE // LANES):
                    s = nrows[p, base, pl.ds(c * LANES, LANES)]
                    for k in range(1, d):
                        s = s + nrows[p, base + k, pl.ds(c * LANES, LANES)]
                    nout[p, r, pl.ds(c * LANES, LANES)] = s
                se = erows[p, base, :]
                for k in range(1, d):
                    se = se + erows[p, base + k, :]
                eout[p, r, :] = se
                return 0

            lax.fori_loop(0, bd, acc_row, 0, unroll=False)

        def issue_o(j, p, b=b, d=d, bd=bd, w0=w0):
            obase = pl.multiple_of((w0 + j) * bd, 8)
            cn = pltpu.async_copy(nout.at[p, pl.ds(0, bd)],
                                  nsum_hbm.at[b, pl.ds(obase, bd)],
                                  osem_n.at[p])
            ce = pltpu.async_copy(eout.at[p, pl.ds(0, bd)],
                                  esum_hbm.at[b, pl.ds(obase, bd)],
                                  osem_e.at[p])
            return cn, ce

        def wait_o(p, bd=bd):
            pltpu.make_async_copy(nout.at[p, pl.ds(0, bd)],
                                  nsum_hbm.at[b, pl.ds(0, bd)],
                                  osem_n.at[p]).wait()
            pltpu.make_async_copy(eout.at[p, pl.ds(0, bd)],
                                  esum_hbm.at[b, pl.ds(0, bd)],
                                  osem_e.at[p]).wait()

        g = issue_g(0, 0)
        gb = issue_g(1, 1)
        wait_g(g)
        acc(0, 0)
        issue_o(0, 0)
        g2 = issue_g(2, 0)
        wait_g(gb)
        acc(1, 1)
        issue_o(1, 1)
        del g, gb, g2

        def pair(i, _):
            j = 2 * i + 2
            issue_g(j + 1, 1)
            pltpu.make_async_copy(node_hbm.at[nidx_v.at[0]], nrows.at[0],
                                  gsem_n.at[0]).wait()
            pltpu.make_async_copy(edge_hbm.at[eidx_v.at[0]], erows.at[0],
                                  gsem_e.at[0]).wait()
            wait_o(0)
            acc(j, 0)
            issue_o(j, 0)
            issue_g(j + 2, 0)
            pltpu.make_async_copy(node_hbm.at[nidx_v.at[0]], nrows.at[1],
                                  gsem_n.at[1]).wait()
            pltpu.make_async_copy(edge_hbm.at[eidx_v.at[0]], erows.at[1],
                                  gsem_e.at[1]).wait()
            wait_o(1)
            acc(j + 1, 1)
            issue_o(j + 1, 1)
            return 0

        lax.fori_loop(0, (per - 3) // 2, pair, 0, unroll=False)

        pltpu.make_async_copy(node_hbm.at[nidx_v.at[0]], nrows.at[0],
                              gsem_n.at[0]).wait()
        pltpu.make_async_copy(edge_hbm.at[eidx_v.at[0]], erows.at[0],
                              gsem_e.at[0]).wait()
        wait_o(0)
        acc(per - 1, 0)
        issue_o(per - 1, 0)
        wait_o(0)
        wait_o(1)


def _sc_gather_sums(node_repr, edge_repr, nidx, eidx):
    mesh = plsc.VectorSubcoreMesh(core_axis_name="c", subcore_axis_name="s")
    out_type = (
        jax.ShapeDtypeStruct((4, GP, NODE_SIZE), jnp.float32),
        jax.ShapeDtypeStruct((4, GP, EDGE_SIZE), jnp.float32),
    )
    max_per = max(PERS)
    scratch = [
        pltpu.VMEM((max_per, BLK), jnp.int32),
        pltpu.VMEM((max_per, BLK), jnp.int32),
        pltpu.VMEM((2, BLK, NODE_SIZE), jnp.float32),
        pltpu.VMEM((2, BLK, EDGE_SIZE), jnp.float32),
        pltpu.VMEM((2, BLK, NODE_SIZE), jnp.float32),
        pltpu.VMEM((2, BLK, EDGE_SIZE), jnp.float32),
        pltpu.SemaphoreType.DMA((2,)),
        pltpu.SemaphoreType.DMA((2,)),
        pltpu.SemaphoreType.DMA((2,)),
        pltpu.SemaphoreType.DMA((2,)),
    ]
    fn = pl.kernel(_sc_body, out_type=out_type, mesh=mesh,
                   scratch_types=scratch,
                   compiler_params=pltpu.CompilerParams(
                       use_tc_tiling_on_sc=False))
    return fn(node_repr, edge_repr,
              nidx[0], eidx[0], nidx[1], eidx[1],
              nidx[2], eidx[2], nidx[3], eidx[3])


R1 = 1000
NB1 = 4 * GROUP // R1
BPB = GROUP // R1

R2 = 2000
NB2 = 4 * GROUP // R2


def _tc_act_body(node_ref, nsum_ref, esum_ref, wself_ref, wn_ref, we_ref,
                 bias_ref, act_ref, s1_ref, s2_ref):
    g = pl.program_id(0)
    act = jnp.dot(node_ref[...], wself_ref[...],
                  preferred_element_type=jnp.float32)
    act = act + jnp.dot(nsum_ref[0], wn_ref[0],
                        preferred_element_type=jnp.float32)
    act = act + jnp.dot(esum_ref[0], we_ref[0],
                        preferred_element_type=jnp.float32)
    act = act + bias_ref[...]
    act_ref[...] = act
    p = jnp.sum(act.reshape(R1 // 8, 8, OUT_SIZE), axis=0)
    q = jnp.sum((act * act).reshape(R1 // 8, 8, OUT_SIZE), axis=0)

    @pl.when(g == 0)
    def _():
        s1_ref[...] = p
        s2_ref[...] = q

    @pl.when(g > 0)
    def _():
        s1_ref[...] += p
        s2_ref[...] += q


def _tc_act(node_repr, nsum, esum, W_self, wn_stack, we_stack, bias):
    return pl.pallas_call(
        _tc_act_body,
        grid=(NB1,),
        in_specs=[
            pl.BlockSpec((R1, NODE_SIZE), lambda g: (g, 0)),
            pl.BlockSpec((1, R1, NODE_SIZE), lambda g: (g // BPB, g % BPB, 0)),
            pl.BlockSpec((1, R1, EDGE_SIZE), lambda g: (g // BPB, g % BPB, 0)),
            pl.BlockSpec((NODE_SIZE, OUT_SIZE), lambda g: (0, 0)),
            pl.BlockSpec((1, NODE_SIZE, OUT_SIZE), lambda g: (g // BPB, 0, 0)),
            pl.BlockSpec((1, EDGE_SIZE, OUT_SIZE), lambda g: (g // BPB, 0, 0)),
            pl.BlockSpec((1, OUT_SIZE), lambda g: (0, 0)),
        ],
        out_specs=[
            pl.BlockSpec((R1, OUT_SIZE), lambda g: (g, 0)),
            pl.BlockSpec((8, OUT_SIZE), lambda g: (0, 0)),
            pl.BlockSpec((8, OUT_SIZE), lambda g: (0, 0)),
        ],
        out_shape=[
            jax.ShapeDtypeStruct((4 * GROUP, OUT_SIZE), jnp.float32),
            jax.ShapeDtypeStruct((8, OUT_SIZE), jnp.float32),
            jax.ShapeDtypeStruct((8, OUT_SIZE), jnp.float32),
        ],
    )(node_repr, nsum, esum, W_self, wn_stack, we_stack, bias)


def _tc_norm_body(act_ref, mu_ref, inv_ref, out_ref):
    out_ref[...] = jnp.maximum(
        (act_ref[...] - mu_ref[...]) * inv_ref[...], 0.0)


def _tc_norm(act, mu, inv):
    return pl.pallas_call(
        _tc_norm_body,
        grid=(NB2,),
        in_specs=[
            pl.BlockSpec((R2, OUT_SIZE), lambda g: (g, 0)),
            pl.BlockSpec((1, OUT_SIZE), lambda g: (0, 0)),
            pl.BlockSpec((1, OUT_SIZE), lambda g: (0, 0)),
        ],
        out_specs=pl.BlockSpec((R2, OUT_SIZE), lambda g: (g, 0)),
        out_shape=jax.ShapeDtypeStruct((4 * GROUP, OUT_SIZE), jnp.float32),
    )(act, mu, inv)


def _prep_idx(a, d):
    nblk = _ceil_div(GROUP * d, BLK)
    f = a.reshape(-1).astype(jnp.int32)
    f = jnp.pad(f, (0, nblk * BLK - f.shape[0]))
    return f.reshape(nblk, BLK)


def kernel(node_repr, edge_repr, nb_node_d1, nb_edge_d1, nb_node_d2,
           nb_edge_d2, nb_node_d4, nb_edge_d4, nb_node_d8, nb_edge_d8,
           W_self, W_deg1, W_deg2, W_deg4, W_deg8, W_deg16, bias):
    nidx = tuple(_prep_idx(a, d) for a, d in
                 zip((nb_node_d1, nb_node_d2, nb_node_d4, nb_node_d8), DEGS))
    eidx = tuple(_prep_idx(a, d) for a, d in
                 zip((nb_edge_d1, nb_edge_d2, nb_edge_d4, nb_edge_d8), DEGS))
    nsum, esum = _sc_gather_sums(node_repr, edge_repr, nidx, eidx)

    wn_stack = jnp.stack([W[:NODE_SIZE] for W in
                          (W_deg1, W_deg2, W_deg4, W_deg8)])
    we_stack = jnp.stack([W[NODE_SIZE:] for W in
                          (W_deg1, W_deg2, W_deg4, W_deg8)])
    act, s1, s2 = _tc_act(node_repr, nsum, esum, W_self, wn_stack, we_stack,
                          bias)
    n = jnp.float32(4 * GROUP)
    mu = jnp.sum(s1, axis=0, keepdims=True) / n
    var = jnp.sum(s2, axis=0, keepdims=True) / n - mu * mu
    inv = lax.rsqrt(var + EPS)
    return _tc_norm(act, mu, inv)


# speedup vs baseline: 2.9508x; 1.0343x over previous
import functools

import jax
import jax.numpy as jnp
from jax import lax
from jax.experimental import pallas as pl
from jax.experimental.pallas import tpu as pltpu
from jax.experimental.pallas import tpu_sc as plsc

N_NODES = 100000
NODE_SIZE = 128
EDGE_SIZE = 16
OUT_SIZE = 128
GROUP = 25000
DEGS = (1, 2, 4, 8)
N_EDGES = 375000
EPS = 1e-5

NC = 2
NS = 16
NW = NC * NS
LANES = 16
BLK = 128


def _ceil_div(a, b):
    return (a + b - 1) // b


NBLKS = tuple(_ceil_div(GROUP * d, BLK) for d in DEGS)
PERS = tuple(_ceil_div(nb, NW) for nb in NBLKS)
GP = max(nb * BLK // d for nb, d in zip(NBLKS, DEGS))


def _sc_body(node_hbm, edge_hbm,
             ni1, ei1, ni2, ei2, ni4, ei4, ni8, ei8,
             nsum_hbm, esum_hbm,
             nidx_v, eidx_v,
             nrows, erows, nout, eout,
             gsem_n, gsem_e, osem_n, osem_e):
    wid = lax.axis_index("s") * NC + lax.axis_index("c")
    nidx_all = (ni1, ni2, ni4, ni8)
    eidx_all = (ei1, ei2, ei4, ei8)

    for b, d in enumerate(DEGS):
        nidx_hbm = nidx_all[b]
        eidx_hbm = eidx_all[b]
        nblk = NBLKS[b]
        per = PERS[b]
        bd = BLK // d

        w0 = jnp.minimum(wid * per, nblk - per)
        pltpu.sync_copy(nidx_hbm.at[pl.ds(w0, per)], nidx_v.at[pl.ds(0, per)])
        pltpu.sync_copy(eidx_hbm.at[pl.ds(w0, per)], eidx_v.at[pl.ds(0, per)])

        def issue_g(j, p):
            cn = pltpu.async_copy(node_hbm.at[nidx_v.at[j]], nrows.at[p],
                                  gsem_n.at[p])
            ce = pltpu.async_copy(edge_hbm.at[eidx_v.at[j]], erows.at[p],
                                  gsem_e.at[p])
            return cn, ce

        def wait_g(cps):
            cps[0].wait()
            cps[1].wait()

        def acc(j, p, d=d, bd=bd):
            def acc_row(r, _):
                base = r * d
                for c in range(NODE_SIZE // LANES):
                    s = nrows[p, base, pl.ds(c * LANES, LANES)]
                    for k in range(1, d):
                        s = s + nrows[p, base + k, pl.ds(c * LANES, LANES)]
                    nout[p, r, pl.ds(c * LANES, LANES)] = s
                se = erows[p, base, :]
                for k in range(1, d):
                    se = se + erows[p, base + k, :]
                eout[p, r, :] = se
                return 0

            lax.fori_loop(0, bd, acc_row, 0, unroll=False)

        def issue_o(j, p, b=b, d=d, bd=bd, w0=w0):
            obase = pl.multiple_of((w0 + j) * bd, 8)
            cn = pltpu.async_copy(nout.at[p, pl.ds(0, bd)],
                                  nsum_hbm.at[b, pl.ds(obase, bd)],
                                  osem_n.at[p])
            ce = pltpu.async_copy(eout.at[p, pl.ds(0, bd)],
                                  esum_hbm.at[b, pl.ds(obase, bd)],
                                  osem_e.at[p])
            return cn, ce

        def wait_o(p, bd=bd):
            pltpu.make_async_copy(nout.at[p, pl.ds(0, bd)],
                                  nsum_hbm.at[b, pl.ds(0, bd)],
                                  osem_n.at[p]).wait()
            pltpu.make_async_copy(eout.at[p, pl.ds(0, bd)],
                                  esum_hbm.at[b, pl.ds(0, bd)],
                                  osem_e.at[p]).wait()

        g = issue_g(0, 0)
        gb = issue_g(1, 1)
        wait_g(g)
        acc(0, 0)
        issue_o(0, 0)
        g2 = issue_g(2, 0)
        wait_g(gb)
        acc(1, 1)
        issue_o(1, 1)
        del g, gb, g2

        def pair(i, _):
            j = 2 * i + 2
            issue_g(j + 1, 1)
            pltpu.make_async_copy(node_hbm.at[nidx_v.at[0]], nrows.at[0],
                                  gsem_n.at[0]).wait()
            pltpu.make_async_copy(edge_hbm.at[eidx_v.at[0]], erows.at[0],
                                  gsem_e.at[0]).wait()
            wait_o(0)
            acc(j, 0)
            issue_o(j, 0)
            issue_g(j + 2, 0)
            pltpu.make_async_copy(node_hbm.at[nidx_v.at[0]], nrows.at[1],
                                  gsem_n.at[1]).wait()
            pltpu.make_async_copy(edge_hbm.at[eidx_v.at[0]], erows.at[1],
                                  gsem_e.at[1]).wait()
            wait_o(1)
            acc(j + 1, 1)
            issue_o(j + 1, 1)
            return 0

        lax.fori_loop(0, (per - 3) // 2, pair, 0, unroll=False)

        pltpu.make_async_copy(node_hbm.at[nidx_v.at[0]], nrows.at[0],
                              gsem_n.at[0]).wait()
        pltpu.make_async_copy(edge_hbm.at[eidx_v.at[0]], erows.at[0],
                              gsem_e.at[0]).wait()
        wait_o(0)
        acc(per - 1, 0)
        issue_o(per - 1, 0)
        wait_o(0)
        wait_o(1)


def _sc_gather_sums(node_repr, edge_repr, nidx, eidx):
    mesh = plsc.VectorSubcoreMesh(core_axis_name="c", subcore_axis_name="s")
    out_type = (
        jax.ShapeDtypeStruct((4, GP, NODE_SIZE), jnp.float32),
        jax.ShapeDtypeStruct((4, GP, EDGE_SIZE), jnp.float32),
    )
    max_per = max(PERS)
    scratch = [
        pltpu.VMEM((max_per, BLK), jnp.int32),
        pltpu.VMEM((max_per, BLK), jnp.int32),
        pltpu.VMEM((2, BLK, NODE_SIZE), jnp.float32),
        pltpu.VMEM((2, BLK, EDGE_SIZE), jnp.float32),
        pltpu.VMEM((2, BLK, NODE_SIZE), jnp.float32),
        pltpu.VMEM((2, BLK, EDGE_SIZE), jnp.float32),
        pltpu.SemaphoreType.DMA((2,)),
        pltpu.SemaphoreType.DMA((2,)),
        pltpu.SemaphoreType.DMA((2,)),
        pltpu.SemaphoreType.DMA((2,)),
    ]
    fn = pl.kernel(_sc_body, out_type=out_type, mesh=mesh,
                   scratch_types=scratch,
                   compiler_params=pltpu.CompilerParams(
                       use_tc_tiling_on_sc=False))
    return fn(node_repr, edge_repr,
              nidx[0], eidx[0], nidx[1], eidx[1],
              nidx[2], eidx[2], nidx[3], eidx[3])


R1 = 1000
NB1 = 4 * GROUP // R1
BPB = GROUP // R1

R2 = 2000
NB2 = 4 * GROUP // R2


def _tc_fused_body(node_ref, nsum_ref, esum_ref, wself_ref, wn_ref, we_ref,
                   bias_ref, out_ref, act_scr, s1_scr, s2_scr, stat_scr):
    p = pl.program_id(0)
    g = pl.program_id(1)

    @pl.when(p == 0)
    def _():
        act = jnp.dot(node_ref[...], wself_ref[...],
                      preferred_element_type=jnp.float32)
        act = act + jnp.dot(nsum_ref[0], wn_ref[0],
                            preferred_element_type=jnp.float32)
        act = act + jnp.dot(esum_ref[0], we_ref[0],
                            preferred_element_type=jnp.float32)
        act = act + bias_ref[...]
        act_scr[pl.ds(g * R1, R1), :] = act
        m1 = jnp.sum(act.reshape(R1 // 8, 8, OUT_SIZE), axis=0)
        m2 = jnp.sum((act * act).reshape(R1 // 8, 8, OUT_SIZE), axis=0)

        @pl.when(g == 0)
        def _():
            s1_scr[...] = m1
            s2_scr[...] = m2

        @pl.when(g > 0)
        def _():
            s1_scr[...] += m1
            s2_scr[...] += m2

    @pl.when(p == 1)
    def _():
        @pl.when(g == 0)
        def _():
            n = jnp.float32(4 * GROUP)
            mu = jnp.sum(s1_scr[...], axis=0, keepdims=True) / n
            var = jnp.sum(s2_scr[...], axis=0, keepdims=True) / n - mu * mu
            stat_scr[0:1, :] = mu
            stat_scr[1:2, :] = lax.rsqrt(var + EPS)

        act = act_scr[pl.ds(g * R1, R1), :]
        out_ref[...] = jnp.maximum(
            (act - stat_scr[0:1, :]) * stat_scr[1:2, :], 0.0)


def _tc_fused(node_repr, nsum, esum, W_self, wn_stack, we_stack, bias):
    return pl.pallas_call(
        _tc_fused_body,
        grid=(2, NB1),
        in_specs=[
            pl.BlockSpec((R1, NODE_SIZE),
                         lambda p, g: (jnp.where(p == 0, g, NB1 - 1), 0)),
            pl.BlockSpec((1, R1, NODE_SIZE),
                         lambda p, g: (jnp.where(p == 0, g // BPB, 3),
                                       jnp.where(p == 0, g % BPB, BPB - 1),
                                       0)),
            pl.BlockSpec((1, R1, EDGE_SIZE),
                         lambda p, g: (jnp.where(p == 0, g // BPB, 3),
                                       jnp.where(p == 0, g % BPB, BPB - 1),
                                       0)),
            pl.BlockSpec((NODE_SIZE, OUT_SIZE), lambda p, g: (0, 0)),
            pl.BlockSpec((1, NODE_SIZE, OUT_SIZE),
                         lambda p, g: (jnp.where(p == 0, g // BPB, 3), 0, 0)),
            pl.BlockSpec((1, EDGE_SIZE, OUT_SIZE),
                         lambda p, g: (jnp.where(p == 0, g // BPB, 3), 0, 0)),
            pl.BlockSpec((1, OUT_SIZE), lambda p, g: (0, 0)),
        ],
        out_specs=pl.BlockSpec((R1, OUT_SIZE),
                               lambda p, g: (jnp.where(p == 1, g, 0), 0)),
        out_shape=jax.ShapeDtypeStruct((4 * GROUP, OUT_SIZE), jnp.float32),
        scratch_shapes=[
            pltpu.VMEM((4 * GROUP, OUT_SIZE), jnp.float32),
            pltpu.VMEM((8, OUT_SIZE), jnp.float32),
            pltpu.VMEM((8, OUT_SIZE), jnp.float32),
            pltpu.VMEM((8, OUT_SIZE), jnp.float32),
        ],
        compiler_params=pltpu.CompilerParams(
            vmem_limit_bytes=128 * 1024 * 1024),
    )(node_repr, nsum, esum, W_self, wn_stack, we_stack, bias)


def _prep_idx(a, d):
    nblk = _ceil_div(GROUP * d, BLK)
    f = a.reshape(-1).astype(jnp.int32)
    f = jnp.pad(f, (0, nblk * BLK - f.shape[0]))
    return f.reshape(nblk, BLK)


def kernel(node_repr, edge_repr, nb_node_d1, nb_edge_d1, nb_node_d2,
           nb_edge_d2, nb_node_d4, nb_edge_d4, nb_node_d8, nb_edge_d8,
           W_self, W_deg1, W_deg2, W_deg4, W_deg8, W_deg16, bias):
    nidx = tuple(_prep_idx(a, d) for a, d in
                 zip((nb_node_d1, nb_node_d2, nb_node_d4, nb_node_d8), DEGS))
    eidx = tuple(_prep_idx(a, d) for a, d in
                 zip((nb_edge_d1, nb_edge_d2, nb_edge_d4, nb_edge_d8), DEGS))
    nsum, esum = _sc_gather_sums(node_repr, edge_repr, nidx, eidx)

    wn_stack = jnp.stack([W[:NODE_SIZE] for W in
                          (W_deg1, W_deg2, W_deg4, W_deg8)])
    we_stack = jnp.stack([W[NODE_SIZE:] for W in
                          (W_deg1, W_deg2, W_deg4, W_deg8)])
    return _tc_fused(node_repr, nsum, esum, W_self, wn_stack, we_stack, bias)


# speedup vs baseline: 3.3315x; 1.1290x over previous
import functools

import jax
import jax.numpy as jnp
from jax import lax
from jax.experimental import pallas as pl
from jax.experimental.pallas import tpu as pltpu
from jax.experimental.pallas import tpu_sc as plsc

N_NODES = 100000
NODE_SIZE = 128
EDGE_SIZE = 16
OUT_SIZE = 128
GROUP = 25000
DEGS = (1, 2, 4, 8)
N_EDGES = 375000
EPS = 1e-5

NC = 2
NS = 16
NW = NC * NS
LANES = 16
BLK = 128


def _ceil_div(a, b):
    return (a + b - 1) // b


NBLKS = tuple(_ceil_div(GROUP * d, BLK) for d in DEGS)
PERS = tuple(_ceil_div(nb, NW) for nb in NBLKS)
MAXPER = max(PERS)


def _make_sc_gather_body(width):
    def body(table_hbm, i1, i2, i3, i4, sum_hbm,
             idx_v, rows, outb, gsem, osem):
        wid = lax.axis_index("s") * NC + lax.axis_index("c")
        idx_all = (i1, i2, i3, i4)
        for b, d in enumerate(DEGS):
            idx_hbm = idx_all[b]
            per = PERS[b]
            bd = BLK // d
            span = per * BLK

            s0 = pl.multiple_of(
                jnp.minimum(wid * span, GROUP * d - span), 8)
            pltpu.sync_copy(idx_hbm.at[pl.ds(s0, span)],
                            idx_v.at[pl.ds(0, span)])
            o0 = s0 // d

            def issue_g(j, p):
                pltpu.async_copy(
                    table_hbm.at[idx_v.at[pl.ds(j * BLK, BLK)]],
                    rows.at[p], gsem.at[p])

            def wait_g(p):
                pltpu.make_async_copy(
                    table_hbm.at[idx_v.at[pl.ds(0, BLK)]],
                    rows.at[p], gsem.at[p]).wait()

            def acc(j, p, d=d, bd=bd):
                if d == 1:
                    def cp_row(r, _):
                        for c in range(width // LANES):
                            outb[p, r, pl.ds(c * LANES, LANES)] = (
                                rows[p, r, pl.ds(c * LANES, LANES)])
                        return 0
                    lax.fori_loop(0, bd, cp_row, 0, unroll=False)
                else:
                    def acc_row(r, _):
                        base = r * d
                        for c in range(width // LANES):
                            s = rows[p, base, pl.ds(c * LANES, LANES)]
                            for k in range(1, d):
                                s = s + rows[p, base + k,
                                             pl.ds(c * LANES, LANES)]
                            outb[p, r, pl.ds(c * LANES, LANES)] = s
                        return 0
                    lax.fori_loop(0, bd, acc_row, 0, unroll=False)

            def issue_o(j, p, b=b, bd=bd, o0=o0):
                obase = pl.multiple_of(o0 + j * bd, 8)
                pltpu.async_copy(
                    outb.at[p, pl.ds(0, bd)],
                    sum_hbm.at[b, pl.ds(obase, bd)], osem.at[p])

            def wait_o(p, b=b, bd=bd):
                pltpu.make_async_copy(
                    outb.at[p, pl.ds(0, bd)],
                    sum_hbm.at[b, pl.ds(0, bd)], osem.at[p]).wait()

            issue_g(0, 0)
            issue_g(1, 1)
            wait_g(0)
            acc(0, 0)
            issue_o(0, 0)
            issue_g(2, 0)
            wait_g(1)
            acc(1, 1)
            issue_o(1, 1)

            def pair(i, _):
                j = 2 * i + 2
                issue_g(j + 1, 1)
                wait_g(0)
                wait_o(0)
                acc(j, 0)
                issue_o(j, 0)
                issue_g(j + 2, 0)
                wait_g(1)
                wait_o(1)
                acc(j + 1, 1)
                issue_o(j + 1, 1)
                return 0

            lax.fori_loop(0, (per - 3) // 2, pair, 0, unroll=False)

            wait_g(0)
            wait_o(0)
            acc(per - 1, 0)
            issue_o(per - 1, 0)
            wait_o(0)
            wait_o(1)

    return body


def _sc_gather_sums(table, idx4, width, tc_tiling):
    mesh = plsc.VectorSubcoreMesh(core_axis_name="c", subcore_axis_name="s")
    out_type = jax.ShapeDtypeStruct((4, GROUP, width), jnp.float32)
    scratch = [
        pltpu.VMEM((MAXPER * BLK,), jnp.int32),
        pltpu.VMEM((2, BLK, width), jnp.float32),
        pltpu.VMEM((2, BLK, width), jnp.float32),
        pltpu.SemaphoreType.DMA((2,)),
        pltpu.SemaphoreType.DMA((2,)),
    ]
    fn = pl.kernel(_make_sc_gather_body(width), out_type=out_type, mesh=mesh,
                   scratch_types=scratch,
                   compiler_params=pltpu.CompilerParams(
                       use_tc_tiling_on_sc=tc_tiling))
    return fn(table, *idx4)


R1 = 1000
NB1 = 4 * GROUP // R1
BPB = GROUP // R1


def _tc_fused_body(node_ref, nsum_ref, esum_ref, wself_ref, wn_ref, we_ref,
                   bias_ref, out_ref, act_scr, s1_scr, s2_scr, stat_scr):
    p = pl.program_id(0)
    g = pl.program_id(1)

    @pl.when(p == 0)
    def _():
        act = jnp.dot(node_ref[...], wself_ref[...],
                      preferred_element_type=jnp.float32)
        act = act + jnp.dot(nsum_ref[0], wn_ref[0],
                            preferred_element_type=jnp.float32)
        act = act + jnp.dot(esum_ref[0], we_ref[0],
                            preferred_element_type=jnp.float32)
        act = act + bias_ref[...]
        act_scr[pl.ds(g * R1, R1), :] = act
        m1 = jnp.sum(act.reshape(R1 // 8, 8, OUT_SIZE), axis=0)
        m2 = jnp.sum((act * act).reshape(R1 // 8, 8, OUT_SIZE), axis=0)

        @pl.when(g == 0)
        def _():
            s1_scr[...] = m1
            s2_scr[...] = m2

        @pl.when(g > 0)
        def _():
            s1_scr[...] += m1
            s2_scr[...] += m2

    @pl.when(p == 1)
    def _():
        @pl.when(g == 0)
        def _():
            n = jnp.float32(4 * GROUP)
            mu = jnp.sum(s1_scr[...], axis=0, keepdims=True) / n
            var = jnp.sum(s2_scr[...], axis=0, keepdims=True) / n - mu * mu
            stat_scr[0:1, :] = mu
            stat_scr[1:2, :] = lax.rsqrt(var + EPS)

        act = act_scr[pl.ds(g * R1, R1), :]
        out_ref[...] = jnp.maximum(
            (act - stat_scr[0:1, :]) * stat_scr[1:2, :], 0.0)


def _tc_fused(node_repr, nsum, esum, W_self, wn_stack, we_stack, bias):
    return pl.pallas_call(
        _tc_fused_body,
        grid=(2, NB1),
        in_specs=[
            pl.BlockSpec((R1, NODE_SIZE),
                         lambda p, g: (jnp.where(p == 0, g, NB1 - 1), 0)),
            pl.BlockSpec((1, R1, NODE_SIZE),
                         lambda p, g: (jnp.where(p == 0, g // BPB, 3),
                                       jnp.where(p == 0, g % BPB, BPB - 1),
                                       0)),
            pl.BlockSpec((1, R1, EDGE_SIZE),
                         lambda p, g: (jnp.where(p == 0, g // BPB, 3),
                                       jnp.where(p == 0, g % BPB, BPB - 1),
                                       0)),
            pl.BlockSpec((NODE_SIZE, OUT_SIZE), lambda p, g: (0, 0)),
            pl.BlockSpec((1, NODE_SIZE, OUT_SIZE),
                         lambda p, g: (jnp.where(p == 0, g // BPB, 3), 0, 0)),
            pl.BlockSpec((1, EDGE_SIZE, OUT_SIZE),
                         lambda p, g: (jnp.where(p == 0, g // BPB, 3), 0, 0)),
            pl.BlockSpec((1, OUT_SIZE), lambda p, g: (0, 0)),
        ],
        out_specs=pl.BlockSpec((R1, OUT_SIZE),
                               lambda p, g: (jnp.where(p == 1, g, 0), 0)),
        out_shape=jax.ShapeDtypeStruct((4 * GROUP, OUT_SIZE), jnp.float32),
        scratch_shapes=[
            pltpu.VMEM((4 * GROUP, OUT_SIZE), jnp.float32),
            pltpu.VMEM((8, OUT_SIZE), jnp.float32),
            pltpu.VMEM((8, OUT_SIZE), jnp.float32),
            pltpu.VMEM((8, OUT_SIZE), jnp.float32),
        ],
        compiler_params=pltpu.CompilerParams(
            vmem_limit_bytes=128 * 1024 * 1024),
    )(node_repr, nsum, esum, W_self, wn_stack, we_stack, bias)


def kernel(node_repr, edge_repr, nb_node_d1, nb_edge_d1, nb_node_d2,
           nb_edge_d2, nb_node_d4, nb_edge_d4, nb_node_d8, nb_edge_d8,
           W_self, W_deg1, W_deg2, W_deg4, W_deg8, W_deg16, bias):
    nidx = tuple(a.reshape(-1).astype(jnp.int32)
                 for a in (nb_node_d1, nb_node_d2, nb_node_d4, nb_node_d8))
    eidx = tuple(a.reshape(-1).astype(jnp.int32)
                 for a in (nb_edge_d1, nb_edge_d2, nb_edge_d4, nb_edge_d8))
    nsum = _sc_gather_sums(node_repr, nidx, NODE_SIZE, True)
    esum = _sc_gather_sums(edge_repr, eidx, EDGE_SIZE, False)

    wn_stack = jnp.stack([W[:NODE_SIZE] for W in
                          (W_deg1, W_deg2, W_deg4, W_deg8)])
    we_stack = jnp.stack([W[NODE_SIZE:] for W in
                          (W_deg1, W_deg2, W_deg4, W_deg8)])
    return _tc_fused(node_repr, nsum, esum, W_self, wn_stack, we_stack, bias)
